# baseline (speedup 1.0000x reference)
import functools

import jax
import jax.numpy as jnp
from jax import lax
from jax.experimental import pallas as pl
from jax.experimental.pallas import tpu as pltpu
from jax.experimental.pallas import tpu_sc as plsc

_CHUNK = 128
_BLOCK_T = 8192
_NBUF = 3


def _sc_gather(tokens_3d, emb_table, n_chunks, chunk):
    info = plsc.get_sparse_core_info()
    nw = info.num_cores * info.num_subcores
    t = nw * n_chunks * chunk
    d = emb_table.shape[1]
    per_w = n_chunks * chunk
    mesh = plsc.VectorSubcoreMesh(core_axis_name="c", subcore_axis_name="s")

    @functools.partial(
        pl.kernel,
        mesh=mesh,
        out_type=jax.ShapeDtypeStruct((t, d), jnp.float32),
        scratch_types=[
            pltpu.VMEM((n_chunks, chunk), jnp.int32),
        ]
        + [pltpu.VMEM((chunk, d), jnp.float32) for _ in range(_NBUF)]
        + [pltpu.SemaphoreType.DMA for _ in range(2 * _NBUF)],
    )
    def gather_kernel(tok_hbm, table_hbm, out_hbm, idx_v, *bufs_sems):
        bufs = bufs_sems[:_NBUF]
        gsems = bufs_sems[_NBUF : 2 * _NBUF]
        wsems = bufs_sems[2 * _NBUF :]
        wid = lax.axis_index("s") * info.num_cores + lax.axis_index("c")
        base = wid * per_w
        pltpu.sync_copy(tok_hbm.at[wid], idx_v)
        gcp = [None] * _NBUF
        wcp = [None] * _NBUF
        n_pre = min(2, n_chunks)
        for c in range(n_pre):
            gcp[c % _NBUF] = pltpu.async_copy(
                table_hbm.at[idx_v.at[c]], bufs[c % _NBUF], gsems[c % _NBUF]
            )
        for c in range(n_chunks):
            s = c % _NBUF
            gcp[s].wait()
            wcp[s] = pltpu.async_copy(
                bufs[s], out_hbm.at[pl.ds(base + c * chunk, chunk)], wsems[s]
            )
            nxt = c + n_pre
            if nxt < n_chunks:
                sn = nxt % _NBUF
                if wcp[sn] is not None:
                    wcp[sn].wait()
                gcp[sn] = pltpu.async_copy(
                    table_hbm.at[idx_v.at[nxt]], bufs[sn], gsems[sn]
                )
        for s in range(_NBUF):
            if wcp[s] is not None:
                wcp[s].wait()

    return gather_kernel(tokens_3d, emb_table)


def _tc_compute(emb, cu_seqlens, W, block_t, t_offset=0, prev=None, final=True):
    t, d = emb.shape
    b = cu_seqlens.shape[0] - 1
    grid = t // block_t

    def body(cu_ref, emb_ref, w_ref, *rest):
        if prev is not None:
            prev_ref, out_ref, acc_ref = rest
        else:
            prev_ref = None
            out_ref, acc_ref = rest
        g = pl.program_id(0)

        @pl.when(g == 0)
        def _init():
            acc_ref[...] = jnp.zeros_like(acc_ref)

        e = emb_ref[...]
        logits = jnp.dot(
            e.astype(jnp.bfloat16),
            w_ref[...].astype(jnp.bfloat16),
            preferred_element_type=jnp.float32,
        )
        gate = 0.5 * jnp.tanh(0.5 * logits) + 0.5
        h = e * gate

        pos = t_offset + g * block_t + lax.broadcasted_iota(jnp.int32, (1, block_t), 1)
        seg = jnp.zeros((1, block_t), jnp.int32)
        for j in range(1, b):
            seg = seg + (pos >= cu_ref[j]).astype(jnp.int32)
        onehot = (lax.broadcasted_iota(jnp.int32, (b, block_t), 0) == seg).astype(
            jnp.bfloat16
        )
        acc_ref[...] += jnp.dot(
            onehot, h.astype(jnp.bfloat16), preferred_element_type=jnp.float32
        )

        @pl.when(g == grid - 1)
        def _fin():
            total = acc_ref[...]
            if prev_ref is not None:
                total = total + prev_ref[...]
            if final:
                rid = lax.broadcasted_iota(jnp.int32, (b, 1), 0)
                lens = jnp.zeros((b, 1), jnp.float32)
                for j in range(b):
                    lens = lens + jnp.where(
                        rid == j, (cu_ref[j + 1] - cu_ref[j]).astype(jnp.float32), 0.0
                    )
                total = total / jnp.maximum(lens, 1.0)
            out_ref[...] = total

    in_specs = [
        pl.BlockSpec((block_t, d), lambda g, cu: (g, 0)),
        pl.BlockSpec((d, d), lambda g, cu: (0, 0)),
    ]
    args = [cu_seqlens, emb, W]
    if prev is not None:
        in_specs.append(pl.BlockSpec((b, d), lambda g, cu: (0, 0)))
        args.append(prev)
    return pl.pallas_call(
        body,
        grid_spec=pltpu.PrefetchScalarGridSpec(
            num_scalar_prefetch=1,
            grid=(grid,),
            in_specs=in_specs,
            out_specs=pl.BlockSpec((b, d), lambda g, cu: (0, 0)),
            scratch_shapes=[pltpu.VMEM((b, d), jnp.float32)],
        ),
        out_shape=jax.ShapeDtypeStruct((b, d), jnp.float32),
    )(*args)


def kernel(tokens, cu_seqlens, emb_table, W):
    t = tokens.shape[0]
    info = plsc.get_sparse_core_info()
    nw = info.num_cores * info.num_subcores
    part = t // 2
    n_chunks = part // (nw * _CHUNK)
    tok0 = lax.slice(tokens, (0,), (part,)).reshape(nw, n_chunks, _CHUNK)
    tok1 = lax.slice(tokens, (part,), (t,)).reshape(nw, n_chunks, _CHUNK)
    emb0 = _sc_gather(tok0, emb_table, n_chunks, _CHUNK)
    emb1 = _sc_gather(tok1, emb_table, n_chunks, _CHUNK)
    p0 = _tc_compute(emb0, cu_seqlens, W, block_t=_BLOCK_T, t_offset=0, final=False)
    return _tc_compute(
        emb1, cu_seqlens, W, block_t=_BLOCK_T, t_offset=part, prev=p0, final=True
    )


# speedup vs baseline: 1.0044x; 1.0044x over previous
import functools

import jax
import jax.numpy as jnp
from jax import lax
from jax.experimental import pallas as pl
from jax.experimental.pallas import tpu as pltpu
from jax.experimental.pallas import tpu_sc as plsc

_CHUNK = 128
_BLOCK_T = 8192
_NBUF = 3


def _sc_gather(tokens_3d, emb_table, n_chunks, chunk):
    info = plsc.get_sparse_core_info()
    nw = info.num_cores * info.num_subcores
    t = nw * n_chunks * chunk
    d = emb_table.shape[1]
    per_w = n_chunks * chunk
    mesh = plsc.VectorSubcoreMesh(core_axis_name="c", subcore_axis_name="s")

    @functools.partial(
        pl.kernel,
        mesh=mesh,
        out_type=jax.ShapeDtypeStruct((t, d), jnp.float32),
        scratch_types=[
            pltpu.VMEM((n_chunks, chunk), jnp.int32),
        ]
        + [pltpu.VMEM((chunk, d), jnp.float32) for _ in range(_NBUF)]
        + [pltpu.SemaphoreType.DMA for _ in range(2 * _NBUF)],
    )
    def gather_kernel(tok_hbm, table_hbm, out_hbm, idx_v, *bufs_sems):
        bufs = bufs_sems[:_NBUF]
        gsems = bufs_sems[_NBUF : 2 * _NBUF]
        wsems = bufs_sems[2 * _NBUF :]
        wid = lax.axis_index("s") * info.num_cores + lax.axis_index("c")
        base = wid * per_w
        pltpu.sync_copy(tok_hbm.at[wid], idx_v)
        gcp = [None] * _NBUF
        wcp = [None] * _NBUF
        n_pre = min(2, n_chunks)
        for c in range(n_pre):
            gcp[c % _NBUF] = pltpu.async_copy(
                table_hbm.at[idx_v.at[c]], bufs[c % _NBUF], gsems[c % _NBUF]
            )
        for c in range(n_chunks):
            s = c % _NBUF
            gcp[s].wait()
            wcp[s] = pltpu.async_copy(
                bufs[s], out_hbm.at[pl.ds(base + c * chunk, chunk)], wsems[s]
            )
            nxt = c + n_pre
            if nxt < n_chunks:
                sn = nxt % _NBUF
                if wcp[sn] is not None:
                    wcp[sn].wait()
                gcp[sn] = pltpu.async_copy(
                    table_hbm.at[idx_v.at[nxt]], bufs[sn], gsems[sn]
                )
        for s in range(_NBUF):
            if wcp[s] is not None:
                wcp[s].wait()

    return gather_kernel(tokens_3d, emb_table)


def _tc_compute(emb, cu_seqlens, W, block_t, t_offset=0, prev=None, final=True):
    t, d = emb.shape
    b = cu_seqlens.shape[0] - 1
    grid = t // block_t

    def body(cu_ref, emb_ref, w_ref, *rest):
        if prev is not None:
            prev_ref, out_ref, acc_ref = rest
        else:
            prev_ref = None
            out_ref, acc_ref = rest
        g = pl.program_id(0)

        @pl.when(g == 0)
        def _init():
            acc_ref[...] = jnp.zeros_like(acc_ref)

        e = emb_ref[...]
        logits = jnp.dot(
            e.astype(jnp.bfloat16),
            w_ref[...].astype(jnp.bfloat16),
            preferred_element_type=jnp.float32,
        )
        gate = 0.5 * jnp.tanh(0.5 * logits) + 0.5
        h = e * gate

        pos = t_offset + g * block_t + lax.broadcasted_iota(jnp.int32, (1, block_t), 1)
        seg = jnp.zeros((1, block_t), jnp.int32)
        for j in range(1, b):
            seg = seg + (pos >= cu_ref[j]).astype(jnp.int32)
        onehot = (lax.broadcasted_iota(jnp.int32, (b, block_t), 0) == seg).astype(
            jnp.bfloat16
        )
        acc_ref[...] += jnp.dot(
            onehot, h.astype(jnp.bfloat16), preferred_element_type=jnp.float32
        )

        @pl.when(g == grid - 1)
        def _fin():
            total = acc_ref[...]
            if prev_ref is not None:
                total = total + prev_ref[...]
            if final:
                rid = lax.broadcasted_iota(jnp.int32, (b, 1), 0)
                lens = jnp.zeros((b, 1), jnp.float32)
                for j in range(b):
                    lens = lens + jnp.where(
                        rid == j, (cu_ref[j + 1] - cu_ref[j]).astype(jnp.float32), 0.0
                    )
                total = total / jnp.maximum(lens, 1.0)
            out_ref[...] = total

    in_specs = [
        pl.BlockSpec((block_t, d), lambda g, cu: (g, 0)),
        pl.BlockSpec((d, d), lambda g, cu: (0, 0)),
    ]
    args = [cu_seqlens, emb, W]
    if prev is not None:
        in_specs.append(pl.BlockSpec((b, d), lambda g, cu: (0, 0)))
        args.append(prev)
    return pl.pallas_call(
        body,
        grid_spec=pltpu.PrefetchScalarGridSpec(
            num_scalar_prefetch=1,
            grid=(grid,),
            in_specs=in_specs,
            out_specs=pl.BlockSpec((b, d), lambda g, cu: (0, 0)),
            scratch_shapes=[pltpu.VMEM((b, d), jnp.float32)],
        ),
        out_shape=jax.ShapeDtypeStruct((b, d), jnp.float32),
    )(*args)


def kernel(tokens, cu_seqlens, emb_table, W):
    t = tokens.shape[0]
    info = plsc.get_sparse_core_info()
    nw = info.num_cores * info.num_subcores
    n_chunks = t // (nw * _CHUNK)
    emb = _sc_gather(tokens.reshape(nw, n_chunks, _CHUNK), emb_table, n_chunks, _CHUNK)
    return _tc_compute(emb, cu_seqlens, W, block_t=_BLOCK_T)


# speedup vs baseline: 1.0174x; 1.0129x over previous
import functools

import jax
import jax.numpy as jnp
from jax import lax
from jax.experimental import pallas as pl
from jax.experimental.pallas import tpu as pltpu
from jax.experimental.pallas import tpu_sc as plsc

_CHUNK = 64
_BLOCK_T = 8192
_NBUF = 6
_NPRE = 3


def _sc_gather(tokens_3d, emb_table, n_chunks, chunk):
    info = plsc.get_sparse_core_info()
    nw = info.num_cores * info.num_subcores
    t = nw * n_chunks * chunk
    d = emb_table.shape[1]
    per_w = n_chunks * chunk
    mesh = plsc.VectorSubcoreMesh(core_axis_name="c", subcore_axis_name="s")

    @functools.partial(
        pl.kernel,
        mesh=mesh,
        out_type=jax.ShapeDtypeStruct((t, d), jnp.float32),
        scratch_types=[
            pltpu.VMEM((n_chunks, chunk), jnp.int32),
        ]
        + [pltpu.VMEM((chunk, d), jnp.float32) for _ in range(_NBUF)]
        + [pltpu.SemaphoreType.DMA for _ in range(2 * _NBUF)],
    )
    def gather_kernel(tok_hbm, table_hbm, out_hbm, idx_v, *bufs_sems):
        bufs = bufs_sems[:_NBUF]
        gsems = bufs_sems[_NBUF : 2 * _NBUF]
        wsems = bufs_sems[2 * _NBUF :]
        wid = lax.axis_index("s") * info.num_cores + lax.axis_index("c")
        base = wid * per_w
        pltpu.sync_copy(tok_hbm.at[wid], idx_v)
        gcp = [None] * _NBUF
        wcp = [None] * _NBUF
        n_pre = min(_NPRE, n_chunks)
        for c in range(n_pre):
            gcp[c % _NBUF] = pltpu.async_copy(
                table_hbm.at[idx_v.at[c]], bufs[c % _NBUF], gsems[c % _NBUF]
            )
        for c in range(n_chunks):
            s = c % _NBUF
            gcp[s].wait()
            wcp[s] = pltpu.async_copy(
                bufs[s], out_hbm.at[pl.ds(base + c * chunk, chunk)], wsems[s]
            )
            nxt = c + n_pre
            if nxt < n_chunks:
                sn = nxt % _NBUF
                if wcp[sn] is not None:
                    wcp[sn].wait()
                gcp[sn] = pltpu.async_copy(
                    table_hbm.at[idx_v.at[nxt]], bufs[sn], gsems[sn]
                )
        for s in range(_NBUF):
            if wcp[s] is not None:
                wcp[s].wait()

    return gather_kernel(tokens_3d, emb_table)


def _tc_compute(emb, cu_seqlens, W, block_t, t_offset=0, prev=None, final=True):
    t, d = emb.shape
    b = cu_seqlens.shape[0] - 1
    grid = t // block_t

    def body(cu_ref, emb_ref, w_ref, *rest):
        if prev is not None:
            prev_ref, out_ref, acc_ref = rest
        else:
            prev_ref = None
            out_ref, acc_ref = rest
        g = pl.program_id(0)

        @pl.when(g == 0)
        def _init():
            acc_ref[...] = jnp.zeros_like(acc_ref)

        e = emb_ref[...]
        logits = jnp.dot(
            e.astype(jnp.bfloat16),
            w_ref[...].astype(jnp.bfloat16),
            preferred_element_type=jnp.float32,
        )
        gate = 0.5 * jnp.tanh(0.5 * logits) + 0.5
        h = e * gate

        pos = t_offset + g * block_t + lax.broadcasted_iota(jnp.int32, (1, block_t), 1)
        seg = jnp.zeros((1, block_t), jnp.int32)
        for j in range(1, b):
            seg = seg + (pos >= cu_ref[j]).astype(jnp.int32)
        onehot = (lax.broadcasted_iota(jnp.int32, (b, block_t), 0) == seg).astype(
            jnp.bfloat16
        )
        acc_ref[...] += jnp.dot(
            onehot, h.astype(jnp.bfloat16), preferred_element_type=jnp.float32
        )

        @pl.when(g == grid - 1)
        def _fin():
            total = acc_ref[...]
            if prev_ref is not None:
                total = total + prev_ref[...]
            if final:
                rid = lax.broadcasted_iota(jnp.int32, (b, 1), 0)
                lens = jnp.zeros((b, 1), jnp.float32)
                for j in range(b):
                    lens = lens + jnp.where(
                        rid == j, (cu_ref[j + 1] - cu_ref[j]).astype(jnp.float32), 0.0
                    )
                total = total / jnp.maximum(lens, 1.0)
            out_ref[...] = total

    in_specs = [
        pl.BlockSpec((block_t, d), lambda g, cu: (g, 0)),
        pl.BlockSpec((d, d), lambda g, cu: (0, 0)),
    ]
    args = [cu_seqlens, emb, W]
    if prev is not None:
        in_specs.append(pl.BlockSpec((b, d), lambda g, cu: (0, 0)))
        args.append(prev)
    return pl.pallas_call(
        body,
        grid_spec=pltpu.PrefetchScalarGridSpec(
            num_scalar_prefetch=1,
            grid=(grid,),
            in_specs=in_specs,
            out_specs=pl.BlockSpec((b, d), lambda g, cu: (0, 0)),
            scratch_shapes=[pltpu.VMEM((b, d), jnp.float32)],
        ),
        out_shape=jax.ShapeDtypeStruct((b, d), jnp.float32),
    )(*args)


def kernel(tokens, cu_seqlens, emb_table, W):
    t = tokens.shape[0]
    info = plsc.get_sparse_core_info()
    nw = info.num_cores * info.num_subcores
    n_chunks = t // (nw * _CHUNK)
    emb = _sc_gather(tokens.reshape(nw, n_chunks, _CHUNK), emb_table, n_chunks, _CHUNK)
    return _tc_compute(emb, cu_seqlens, W, block_t=_BLOCK_T)


# speedup vs baseline: 1.0326x; 1.0150x over previous
import functools

import jax
import jax.numpy as jnp
from jax import lax
from jax.experimental import pallas as pl
from jax.experimental.pallas import tpu as pltpu
from jax.experimental.pallas import tpu_sc as plsc

_CHUNK = 64
_BLOCK_T = 8192
_NBUF = 6
_NPRE = 4


def _sc_gather(tokens_3d, emb_table, n_chunks, chunk):
    info = plsc.get_sparse_core_info()
    nw = info.num_cores * info.num_subcores
    t = nw * n_chunks * chunk
    d = emb_table.shape[1]
    per_w = n_chunks * chunk
    mesh = plsc.VectorSubcoreMesh(core_axis_name="c", subcore_axis_name="s")

    @functools.partial(
        pl.kernel,
        mesh=mesh,
        out_type=jax.ShapeDtypeStruct((t, d), jnp.float32),
        scratch_types=[
            pltpu.VMEM((n_chunks, chunk), jnp.int32),
        ]
        + [pltpu.VMEM((chunk, d), jnp.float32) for _ in range(_NBUF)]
        + [pltpu.SemaphoreType.DMA for _ in range(2 * _NBUF)],
    )
    def gather_kernel(tok_hbm, table_hbm, out_hbm, idx_v, *bufs_sems):
        bufs = bufs_sems[:_NBUF]
        gsems = bufs_sems[_NBUF : 2 * _NBUF]
        wsems = bufs_sems[2 * _NBUF :]
        wid = lax.axis_index("s") * info.num_cores + lax.axis_index("c")
        base = wid * per_w
        pltpu.sync_copy(tok_hbm.at[wid], idx_v)
        gcp = [None] * _NBUF
        wcp = [None] * _NBUF
        n_pre = min(_NPRE, n_chunks)
        for c in range(n_pre):
            gcp[c % _NBUF] = pltpu.async_copy(
                table_hbm.at[idx_v.at[c]], bufs[c % _NBUF], gsems[c % _NBUF]
            )
        for c in range(n_chunks):
            s = c % _NBUF
            gcp[s].wait()
            wcp[s] = pltpu.async_copy(
                bufs[s], out_hbm.at[pl.ds(base + c * chunk, chunk)], wsems[s]
            )
            nxt = c + n_pre
            if nxt < n_chunks:
                sn = nxt % _NBUF
                if wcp[sn] is not None:
                    wcp[sn].wait()
                gcp[sn] = pltpu.async_copy(
                    table_hbm.at[idx_v.at[nxt]], bufs[sn], gsems[sn]
                )
        for s in range(_NBUF):
            if wcp[s] is not None:
                wcp[s].wait()

    return gather_kernel(tokens_3d, emb_table)


def _tc_compute(emb, cu_seqlens, W, block_t, t_offset=0, prev=None, final=True):
    t, d = emb.shape
    b = cu_seqlens.shape[0] - 1
    grid = t // block_t

    def body(cu_ref, emb_ref, w_ref, *rest):
        if prev is not None:
            prev_ref, out_ref, acc_ref = rest
        else:
            prev_ref = None
            out_ref, acc_ref = rest
        g = pl.program_id(0)

        @pl.when(g == 0)
        def _init():
            acc_ref[...] = jnp.zeros_like(acc_ref)

        e = emb_ref[...]
        logits = jnp.dot(
            e.astype(jnp.bfloat16),
            w_ref[...].astype(jnp.bfloat16),
            preferred_element_type=jnp.float32,
        )
        gate = 0.5 * jnp.tanh(0.5 * logits) + 0.5
        h = e * gate

        pos = t_offset + g * block_t + lax.broadcasted_iota(jnp.int32, (1, block_t), 1)
        seg = jnp.zeros((1, block_t), jnp.int32)
        for j in range(1, b):
            seg = seg + (pos >= cu_ref[j]).astype(jnp.int32)
        onehot = (lax.broadcasted_iota(jnp.int32, (b, block_t), 0) == seg).astype(
            jnp.bfloat16
        )
        acc_ref[...] += jnp.dot(
            onehot, h.astype(jnp.bfloat16), preferred_element_type=jnp.float32
        )

        @pl.when(g == grid - 1)
        def _fin():
            total = acc_ref[...]
            if prev_ref is not None:
                total = total + prev_ref[...]
            if final:
                rid = lax.broadcasted_iota(jnp.int32, (b, 1), 0)
                lens = jnp.zeros((b, 1), jnp.float32)
                for j in range(b):
                    lens = lens + jnp.where(
                        rid == j, (cu_ref[j + 1] - cu_ref[j]).astype(jnp.float32), 0.0
                    )
                total = total / jnp.maximum(lens, 1.0)
            out_ref[...] = total

    in_specs = [
        pl.BlockSpec((block_t, d), lambda g, cu: (g, 0)),
        pl.BlockSpec((d, d), lambda g, cu: (0, 0)),
    ]
    args = [cu_seqlens, emb, W]
    if prev is not None:
        in_specs.append(pl.BlockSpec((b, d), lambda g, cu: (0, 0)))
        args.append(prev)
    return pl.pallas_call(
        body,
        grid_spec=pltpu.PrefetchScalarGridSpec(
            num_scalar_prefetch=1,
            grid=(grid,),
            in_specs=in_specs,
            out_specs=pl.BlockSpec((b, d), lambda g, cu: (0, 0)),
            scratch_shapes=[pltpu.VMEM((b, d), jnp.float32)],
        ),
        out_shape=jax.ShapeDtypeStruct((b, d), jnp.float32),
    )(*args)


def kernel(tokens, cu_seqlens, emb_table, W):
    t = tokens.shape[0]
    info = plsc.get_sparse_core_info()
    nw = info.num_cores * info.num_subcores
    n_chunks = t // (nw * _CHUNK)
    emb = _sc_gather(tokens.reshape(nw, n_chunks, _CHUNK), emb_table, n_chunks, _CHUNK)
    return _tc_compute(emb, cu_seqlens, W, block_t=_BLOCK_T)
